# baseline (speedup 1.0000x reference)
import functools

import jax
import jax.numpy as jnp
from jax import lax
from jax.experimental import pallas as pl
from jax.experimental.pallas import tpu as pltpu
from jax.experimental.pallas import tpu_sc as plsc

_N = 10000
_D = 128
_E = 320000

_NC = 2
_NS = 16
_NW = _NC * _NS
_CHUNK = 128
_NCHUNK = 79
_CH0 = 108
_CH1 = 2 * _NCHUNK - _CH0
_EPW = _NCHUNK * _CHUNK
_E_PAD = _NCHUNK * _CHUNK * _NW
_N_PAD = 10112
_RPT = _N_PAD // _NS
_DW = 16

_mesh = plsc.VectorSubcoreMesh(core_axis_name="c", subcore_axis_name="s")


@functools.partial(
    pl.kernel,
    mesh=_mesh,
    out_type=jax.ShapeDtypeStruct((_NC, _N_PAD, _D), jnp.float32),
    scratch_types=[
        pltpu.VMEM((_CHUNK,), jnp.int32),
        pltpu.VMEM((_CHUNK,), jnp.int32),
        pltpu.VMEM((_CHUNK, _D), jnp.float32),
        pltpu.VMEM_SHARED((_N_PAD, _D), jnp.float32),
        pltpu.SemaphoreType.DMA,
    ],
)
def _sc_segsum(h_hbm, src_hbm, dst_hbm, zeros_hbm, out_hbm,
               src_v, dst_v, rows_v, acc_sh, sem):
    c = lax.axis_index("c")
    s = lax.axis_index("s")
    wid = s * _NC + c
    pltpu.sync_copy(zeros_hbm.at[pl.ds(s * _RPT, _RPT)],
                    acc_sh.at[pl.ds(s * _RPT, _RPT)])
    plsc.subcore_barrier()
    cbase = jnp.where(c == 0, s * _CH0, _NS * _CH0 + s * _CH1)
    nch = jnp.where(c == 0, _CH0, _CH1)

    def body(i, carry):
        base = (cbase + i) * _CHUNK
        pltpu.sync_copy(src_hbm.at[pl.ds(base, _CHUNK)], src_v)
        pltpu.sync_copy(dst_hbm.at[pl.ds(base, _CHUNK)], dst_v)
        pltpu.async_copy(h_hbm.at[src_v], rows_v, sem).wait()
        pltpu.sync_copy(rows_v, acc_sh.at[dst_v], add=True)
        return carry

    lax.fori_loop(0, nch, body, 0)
    plsc.subcore_barrier()
    pltpu.sync_copy(acc_sh.at[pl.ds(s * _RPT, _RPT)],
                    out_hbm.at[c, pl.ds(s * _RPT, _RPT)])


@functools.partial(
    pl.kernel,
    mesh=_mesh,
    out_type=jax.ShapeDtypeStruct((_NC, _N_PAD, _D), jnp.float32),
    scratch_types=[
        pltpu.VMEM((_CHUNK,), jnp.int32),
        pltpu.VMEM((_CHUNK, _D), jnp.float32),
        pltpu.VMEM_SHARED((_N_PAD, _D), jnp.float32),
    ],
)
def _sc_degree(ones_hbm, dst_hbm, zeros_hbm, out_hbm, dst_v, ones_v, acc_sh):
    c = lax.axis_index("c")
    s = lax.axis_index("s")
    wid = s * _NC + c
    pltpu.sync_copy(ones_hbm, ones_v)
    pltpu.sync_copy(zeros_hbm.at[pl.ds(s * _RPT, _RPT)],
                    acc_sh.at[pl.ds(s * _RPT, _RPT)])
    plsc.subcore_barrier()
    wbase = wid * _EPW

    def body(i, carry):
        base = wbase + i * _CHUNK
        pltpu.sync_copy(dst_hbm.at[pl.ds(base, _CHUNK)], dst_v)
        pltpu.sync_copy(ones_v, acc_sh.at[dst_v], add=True)
        return carry

    lax.fori_loop(0, _NCHUNK, body, 0)
    plsc.subcore_barrier()
    pltpu.sync_copy(acc_sh.at[pl.ds(s * _RPT, _RPT)],
                    out_hbm.at[c, pl.ds(s * _RPT, _RPT)])


def _matmuls(agg, h, wl, bl, wr):
    dn = (((1,), (1,)), ((), ()))
    z = lax.dot_general(agg, wl, dn, preferred_element_type=jnp.float32)
    z = z + bl
    z = z + lax.dot_general(h, wr, dn, preferred_element_type=jnp.float32)
    return z


def _tc_layer1_body(p_ref, dp_ref, x_ref, wl_ref, bl_ref, wr_ref,
                    g_ref, be_ref, h_ref, inv_ref):
    deg = dp_ref[0, :_N, :] + dp_ref[1, :_N, :]
    inv = 1.0 / jnp.clip(deg, 1.0, None)
    agg = (p_ref[0, :_N, :] + p_ref[1, :_N, :]) * inv[:, 0:1]
    z = _matmuls(agg, x_ref[...], wl_ref[...], bl_ref[...], wr_ref[...])
    mean = jnp.mean(z, axis=0, keepdims=True)
    var = jnp.mean((z - mean) ** 2, axis=0, keepdims=True)
    zn = (z - mean) / jnp.sqrt(var + 1e-5) * g_ref[...] + be_ref[...]
    h_ref[...] = jnp.maximum(zn, 0.0)
    inv_ref[...] = inv[:, :_DW]


def _tc_layer2_body(p_ref, inv_ref, h_in_ref, wl_ref, bl_ref, wr_ref,
                    g_ref, be_ref, h_ref):
    inv = inv_ref[...]
    agg = (p_ref[0, :_N, :] + p_ref[1, :_N, :]) * inv[:, 0:1]
    z = _matmuls(agg, h_in_ref[...], wl_ref[...], bl_ref[...], wr_ref[...])
    mean = jnp.mean(z, axis=0, keepdims=True)
    var = jnp.mean((z - mean) ** 2, axis=0, keepdims=True)
    zn = (z - mean) / jnp.sqrt(var + 1e-5) * g_ref[...] + be_ref[...]
    h_ref[...] = jnp.maximum(zn, 0.0)


def _tc_layer3_body(p_ref, inv_ref, h_in_ref, wl_ref, bl_ref, wr_ref, out_ref):
    inv = inv_ref[...]
    agg = (p_ref[0, :_N, :] + p_ref[1, :_N, :]) * inv[:, 0:1]
    out_ref[...] = _matmuls(agg, h_in_ref[...], wl_ref[...], bl_ref[...],
                            wr_ref[...])


def kernel(x, edge_index, W1l, b1l, W1r, g1, be1,
           W2l, b2l, W2r, g2, be2, W3l, b3l, W3r):
    ei = edge_index.astype(jnp.int32)
    pad = _E_PAD - _E
    src = jnp.concatenate([ei[0], jnp.zeros((pad,), jnp.int32)])
    dummy = _N + jnp.arange(pad, dtype=jnp.int32) % (_N_PAD - _N)
    dst = jnp.concatenate([ei[1], dummy])
    zeros_d = jnp.zeros((_N_PAD, _D), jnp.float32)
    ones_c = jnp.ones((_CHUNK, _D), jnp.float32)

    def seg(h):
        return _sc_segsum(h, src, dst, zeros_d)

    degp = _sc_degree(ones_c, dst, zeros_d)

    def dense1(p, dp):
        return pl.pallas_call(
            _tc_layer1_body,
            out_shape=[jax.ShapeDtypeStruct((_N, _D), jnp.float32),
                       jax.ShapeDtypeStruct((_N, _DW), jnp.float32)],
        )(p, dp, x, W1l, b1l.reshape(1, _D), W1r,
          g1.reshape(1, _D), be1.reshape(1, _D))

    def dense2(p, inv, h_in):
        return pl.pallas_call(
            _tc_layer2_body,
            out_shape=jax.ShapeDtypeStruct((_N, _D), jnp.float32),
        )(p, inv, h_in, W2l, b2l.reshape(1, _D), W2r,
          g2.reshape(1, _D), be2.reshape(1, _D))

    def dense3(p, inv, h_in):
        return pl.pallas_call(
            _tc_layer3_body,
            out_shape=jax.ShapeDtypeStruct((_N, _D), jnp.float32),
        )(p, inv, h_in, W3l, b3l.reshape(1, _D), W3r)

    p1 = seg(x)
    h1, inv = dense1(p1, degp)
    p2 = seg(h1)
    h2 = dense2(p2, inv, h1)
    p3 = seg(h2)
    return dense3(p3, inv, h2)


# speedup vs baseline: 1.0087x; 1.0087x over previous
import functools

import jax
import jax.numpy as jnp
from jax import lax
from jax.experimental import pallas as pl
from jax.experimental.pallas import tpu as pltpu
from jax.experimental.pallas import tpu_sc as plsc

_N = 10000
_D = 128
_E = 320000

_NC = 2
_NS = 16
_NW = _NC * _NS
_CHUNK = 128
_NCHUNK = 79
_CH0 = 104
_CH1 = 2 * _NCHUNK - _CH0
_EPW = _NCHUNK * _CHUNK
_E_PAD = _NCHUNK * _CHUNK * _NW
_N_PAD = 10112
_RPT = _N_PAD // _NS
_DW = 16

_mesh = plsc.VectorSubcoreMesh(core_axis_name="c", subcore_axis_name="s")


@functools.partial(
    pl.kernel,
    mesh=_mesh,
    out_type=jax.ShapeDtypeStruct((_NC, _N_PAD, _D), jnp.float32),
    scratch_types=[
        pltpu.VMEM((_CHUNK,), jnp.int32),
        pltpu.VMEM((_CHUNK,), jnp.int32),
        pltpu.VMEM((_CHUNK, _D), jnp.float32),
        pltpu.VMEM_SHARED((_N_PAD, _D), jnp.float32),
        pltpu.SemaphoreType.DMA,
    ],
)
def _sc_segsum(h_hbm, src_hbm, dst_hbm, zeros_hbm, out_hbm,
               src_v, dst_v, rows_v, acc_sh, sem):
    c = lax.axis_index("c")
    s = lax.axis_index("s")
    wid = s * _NC + c
    pltpu.sync_copy(zeros_hbm.at[pl.ds(s * _RPT, _RPT)],
                    acc_sh.at[pl.ds(s * _RPT, _RPT)])
    plsc.subcore_barrier()
    cbase = jnp.where(c == 0, s * _CH0, _NS * _CH0 + s * _CH1)
    nch = jnp.where(c == 0, _CH0, _CH1)

    def body(i, carry):
        base = (cbase + i) * _CHUNK
        pltpu.sync_copy(src_hbm.at[pl.ds(base, _CHUNK)], src_v)
        pltpu.sync_copy(dst_hbm.at[pl.ds(base, _CHUNK)], dst_v)
        pltpu.async_copy(h_hbm.at[src_v], rows_v, sem).wait()
        pltpu.sync_copy(rows_v, acc_sh.at[dst_v], add=True)
        return carry

    lax.fori_loop(0, nch, body, 0)
    plsc.subcore_barrier()
    pltpu.sync_copy(acc_sh.at[pl.ds(s * _RPT, _RPT)],
                    out_hbm.at[c, pl.ds(s * _RPT, _RPT)])


@functools.partial(
    pl.kernel,
    mesh=_mesh,
    out_type=jax.ShapeDtypeStruct((_NC, _N_PAD, _D), jnp.float32),
    scratch_types=[
        pltpu.VMEM((_CHUNK,), jnp.int32),
        pltpu.VMEM((_CHUNK, _D), jnp.float32),
        pltpu.VMEM_SHARED((_N_PAD, _D), jnp.float32),
    ],
)
def _sc_degree(ones_hbm, dst_hbm, zeros_hbm, out_hbm, dst_v, ones_v, acc_sh):
    c = lax.axis_index("c")
    s = lax.axis_index("s")
    wid = s * _NC + c
    pltpu.sync_copy(ones_hbm, ones_v)
    pltpu.sync_copy(zeros_hbm.at[pl.ds(s * _RPT, _RPT)],
                    acc_sh.at[pl.ds(s * _RPT, _RPT)])
    plsc.subcore_barrier()
    wbase = wid * _EPW

    def body(i, carry):
        base = wbase + i * _CHUNK
        pltpu.sync_copy(dst_hbm.at[pl.ds(base, _CHUNK)], dst_v)
        pltpu.sync_copy(ones_v, acc_sh.at[dst_v], add=True)
        return carry

    lax.fori_loop(0, _NCHUNK, body, 0)
    plsc.subcore_barrier()
    pltpu.sync_copy(acc_sh.at[pl.ds(s * _RPT, _RPT)],
                    out_hbm.at[c, pl.ds(s * _RPT, _RPT)])


def _matmuls(agg, h, wl, bl, wr):
    dn = (((1,), (1,)), ((), ()))
    z = lax.dot_general(agg, wl, dn, preferred_element_type=jnp.float32)
    z = z + bl
    z = z + lax.dot_general(h, wr, dn, preferred_element_type=jnp.float32)
    return z


def _tc_layer1_body(p_ref, dp_ref, x_ref, wl_ref, bl_ref, wr_ref,
                    g_ref, be_ref, h_ref, inv_ref):
    deg = dp_ref[0, :_N, :] + dp_ref[1, :_N, :]
    inv = 1.0 / jnp.clip(deg, 1.0, None)
    agg = (p_ref[0, :_N, :] + p_ref[1, :_N, :]) * inv[:, 0:1]
    z = _matmuls(agg, x_ref[...], wl_ref[...], bl_ref[...], wr_ref[...])
    mean = jnp.mean(z, axis=0, keepdims=True)
    var = jnp.mean((z - mean) ** 2, axis=0, keepdims=True)
    zn = (z - mean) / jnp.sqrt(var + 1e-5) * g_ref[...] + be_ref[...]
    h_ref[...] = jnp.maximum(zn, 0.0)
    inv_ref[...] = inv[:, :_DW]


def _tc_layer2_body(p_ref, inv_ref, h_in_ref, wl_ref, bl_ref, wr_ref,
                    g_ref, be_ref, h_ref):
    inv = inv_ref[...]
    agg = (p_ref[0, :_N, :] + p_ref[1, :_N, :]) * inv[:, 0:1]
    z = _matmuls(agg, h_in_ref[...], wl_ref[...], bl_ref[...], wr_ref[...])
    mean = jnp.mean(z, axis=0, keepdims=True)
    var = jnp.mean((z - mean) ** 2, axis=0, keepdims=True)
    zn = (z - mean) / jnp.sqrt(var + 1e-5) * g_ref[...] + be_ref[...]
    h_ref[...] = jnp.maximum(zn, 0.0)


def _tc_layer3_body(p_ref, inv_ref, h_in_ref, wl_ref, bl_ref, wr_ref, out_ref):
    inv = inv_ref[...]
    agg = (p_ref[0, :_N, :] + p_ref[1, :_N, :]) * inv[:, 0:1]
    out_ref[...] = _matmuls(agg, h_in_ref[...], wl_ref[...], bl_ref[...],
                            wr_ref[...])


def kernel(x, edge_index, W1l, b1l, W1r, g1, be1,
           W2l, b2l, W2r, g2, be2, W3l, b3l, W3r):
    ei = edge_index.astype(jnp.int32)
    pad = _E_PAD - _E
    src = jnp.concatenate([ei[0], jnp.zeros((pad,), jnp.int32)])
    dummy = _N + jnp.arange(pad, dtype=jnp.int32) % (_N_PAD - _N)
    dst = jnp.concatenate([ei[1], dummy])
    zeros_d = jnp.zeros((_N_PAD, _D), jnp.float32)
    ones_c = jnp.ones((_CHUNK, _D), jnp.float32)

    def seg(h):
        return _sc_segsum(h, src, dst, zeros_d)

    degp = _sc_degree(ones_c, dst, zeros_d)

    def dense1(p, dp):
        return pl.pallas_call(
            _tc_layer1_body,
            out_shape=[jax.ShapeDtypeStruct((_N, _D), jnp.float32),
                       jax.ShapeDtypeStruct((_N, _DW), jnp.float32)],
        )(p, dp, x, W1l, b1l.reshape(1, _D), W1r,
          g1.reshape(1, _D), be1.reshape(1, _D))

    def dense2(p, inv, h_in):
        return pl.pallas_call(
            _tc_layer2_body,
            out_shape=jax.ShapeDtypeStruct((_N, _D), jnp.float32),
        )(p, inv, h_in, W2l, b2l.reshape(1, _D), W2r,
          g2.reshape(1, _D), be2.reshape(1, _D))

    def dense3(p, inv, h_in):
        return pl.pallas_call(
            _tc_layer3_body,
            out_shape=jax.ShapeDtypeStruct((_N, _D), jnp.float32),
        )(p, inv, h_in, W3l, b3l.reshape(1, _D), W3r)

    p1 = seg(x)
    h1, inv = dense1(p1, degp)
    p2 = seg(h1)
    h2 = dense2(p2, inv, h1)
    p3 = seg(h2)
    return dense3(p3, inv, h2)
